# baseline (speedup 1.0000x reference)
import functools

import jax
import jax.numpy as jnp
from jax import lax
from jax.experimental import pallas as pl
from jax.experimental.pallas import tpu as pltpu
from jax.experimental.pallas import tpu_sc as plsc

N_ENT = 10000
N_EDGES = 320000
FEAT = 128

NC = 2
NS = 16
NW = NC * NS
CHUNK = 32
N_CHUNKS = 316
EPW = CHUNK * N_CHUNKS
E_PAD = EPW * NW
AGG_ROWS = 10112
ROWS_PER_TILE = AGG_ROWS // NS
DEG_ROWS = 80
OUT_ROWS = AGG_ROWS + NS * DEG_ROWS


def _sc_body(src_hbm, dst_hbm, et_hbm, feat_hbm, rel_hbm, out_hbm,
             src_v0, src_v1, dst_v0, dst_v1, et_v0, et_v1,
             feat_v0, feat_v1, rel_v0, rel_v1, deg_v, agg_sh,
             sem_f0, sem_f1, sem_r0, sem_r1, sem_s0, sem_s1):
    cid = lax.axis_index("c")
    sid = lax.axis_index("s")
    w = cid * NS + sid

    src_v = (src_v0, src_v1)
    dst_v = (dst_v0, dst_v1)
    et_v = (et_v0, et_v1)
    feat_v = (feat_v0, feat_v1)
    rel_v = (rel_v0, rel_v1)
    sem_f = (sem_f0, sem_f1)
    sem_r = (sem_r0, sem_r1)
    sem_s = (sem_s0, sem_s1)

    zero16f = jnp.zeros((16,), jnp.float32)

    def zrow(r, _):
        for c in range(FEAT // 16):
            feat_v0[r, pl.ds(c * 16, 16)] = zero16f
        return 0
    lax.fori_loop(0, CHUNK, zrow, 0)
    base_row = sid * ROWS_PER_TILE
    _nz = ROWS_PER_TILE // CHUNK
    for k in range(_nz):
        pltpu.sync_copy(feat_v0, agg_sh.at[pl.ds(base_row + k * CHUNK, CHUNK), :])
    if ROWS_PER_TILE % CHUNK:
        pltpu.sync_copy(feat_v0.at[pl.ds(0, ROWS_PER_TILE % CHUNK), :],
                        agg_sh.at[pl.ds(base_row + _nz * CHUNK,
                                        ROWS_PER_TILE % CHUNK), :])

    def zdeg(r, _):
        for c in range(FEAT // 16):
            deg_v[r, pl.ds(c * 16, 16)] = zero16f
        return 0
    lax.fori_loop(0, DEG_ROWS, zdeg, 0)

    plsc.subcore_barrier()

    ebase = w * EPW

    def stage_in(i, p):
        base = ebase + i * CHUNK
        pltpu.sync_copy(src_hbm.at[pl.ds(base, CHUNK)], src_v[p])
        pltpu.sync_copy(et_hbm.at[pl.ds(base, CHUNK)], et_v[p])
        pltpu.sync_copy(dst_hbm.at[pl.ds(base, CHUNK)], dst_v[p])
        pltpu.async_copy(feat_hbm.at[src_v[p]], feat_v[p], sem_f[p])
        pltpu.async_copy(rel_hbm.at[et_v[p]], rel_v[p], sem_r[p])

    def wait_gathers(p):
        pltpu.make_async_copy(feat_hbm.at[src_v[p]], feat_v[p], sem_f[p]).wait()
        pltpu.make_async_copy(rel_hbm.at[et_v[p]], rel_v[p], sem_r[p]).wait()

    def wait_scatter(p):
        pltpu.make_async_copy(feat_v[p], agg_sh.at[dst_v[p]], sem_s[p]).wait()

    def work(i, p):
        wait_gathers(p)

        def mrow(j, _):
            r = j * 4
            for dr in range(4):
                for c in range(FEAT // 16):
                    feat_v[p][r + dr, pl.ds(c * 16, 16)] = (
                        feat_v[p][r + dr, pl.ds(c * 16, 16)]
                        * rel_v[p][r + dr, pl.ds(c * 16, 16)])
            return 0
        lax.fori_loop(0, CHUNK // 4, mrow, 0)

        pltpu.async_copy(feat_v[p], agg_sh.at[dst_v[p]], sem_s[p], add=True)

        def drow(j, _):
            d16 = dst_v[p][pl.ds(j * 16, 16)]
            cnt, last = plsc.scan_count(d16)
            plsc.addupdate_scatter(
                deg_v,
                [lax.shift_right_logical(d16, 7),
                 lax.bitwise_and(d16, 127)],
                cnt.astype(jnp.float32), mask=last)
            return 0
        lax.fori_loop(0, CHUNK // 16, drow, 0)

    stage_in(0, 0)

    def outer(j, _):
        i0 = j * 2
        for p in (0, 1):
            i = i0 + p
            q = 1 - p

            @pl.when(i > 0)
            def _():
                wait_scatter(q)

            @pl.when(i < N_CHUNKS - 1)
            def _():
                stage_in(i + 1, q)

            work(i, p)
        return 0

    lax.fori_loop(0, N_CHUNKS // 2, outer, 0)
    wait_scatter((N_CHUNKS - 1) % 2)

    plsc.subcore_barrier()

    pltpu.sync_copy(agg_sh.at[pl.ds(base_row, ROWS_PER_TILE), :],
                    out_hbm.at[cid, pl.ds(base_row, ROWS_PER_TILE), :])

    plsc.subcore_barrier()

    stage = sid * DEG_ROWS
    pltpu.sync_copy(deg_v, agg_sh.at[pl.ds(stage, DEG_ROWS), :])
    pltpu.sync_copy(agg_sh.at[pl.ds(stage, DEG_ROWS), :],
                    out_hbm.at[cid, pl.ds(AGG_ROWS + stage, DEG_ROWS), :])


_sc_aggregate = functools.partial(
    pl.kernel,
    out_type=jax.ShapeDtypeStruct((NC, OUT_ROWS, FEAT), jnp.float32),
    mesh=plsc.VectorSubcoreMesh(core_axis_name="c", subcore_axis_name="s"),
    compiler_params=pltpu.CompilerParams(needs_layout_passes=False),
    scratch_types=(
        [pltpu.VMEM((CHUNK,), jnp.int32)] * 6
        + [pltpu.VMEM((CHUNK, FEAT), jnp.float32)] * 4
        + [pltpu.VMEM((DEG_ROWS, FEAT), jnp.float32),
           pltpu.VMEM_SHARED((AGG_ROWS, FEAT), jnp.float32)]
        + [pltpu.SemaphoreType.DMA] * 6
    ),
)(_sc_body)


def _tc_body(part_ref, degs_ref, Wg_ref, bg_ref, W1_ref, b1_ref, g_ref,
             be_ref, W2_ref, b2_ref, out_ref):
    agg = part_ref[0, :N_ENT, :] + part_ref[1, :N_ENT, :]
    deg = jnp.sum(degs_ref[...], axis=1, keepdims=True)[:N_ENT]
    h = agg / jnp.maximum(deg, 1.0)
    h = jnp.dot(h, Wg_ref[...], preferred_element_type=jnp.float32,
                precision=lax.Precision.HIGHEST) + bg_ref[...]
    z = jnp.dot(h, W1_ref[...], preferred_element_type=jnp.float32,
                precision=lax.Precision.HIGHEST) + b1_ref[...]
    mean = jnp.mean(z, axis=0, keepdims=True)
    var = jnp.mean(jnp.square(z - mean), axis=0, keepdims=True)
    zn = g_ref[...] * (z - mean) / jnp.sqrt(var + 1e-5) + be_ref[...]
    zr = jnp.maximum(zn, 0.0)
    out_ref[...] = jnp.dot(zr, W2_ref[...], preferred_element_type=jnp.float32,
                           precision=lax.Precision.HIGHEST) + b2_ref[...]


def _tc_dense(partial, degs_t, W_gcn, b_gcn, Wp1, bp1, gamma, beta, Wp2, bp2):
    return pl.pallas_call(
        _tc_body,
        out_shape=jax.ShapeDtypeStruct((N_ENT, FEAT), jnp.float32),
    )(partial, degs_t, W_gcn, b_gcn, Wp1, bp1, gamma, beta, Wp2, bp2)


def kernel(initial_features, relation_embeddings, W_gcn, b_gcn, Wp1, bp1,
           gamma, beta, Wp2, bp2, edge_index, edge_type):
    src = edge_index[0].astype(jnp.int32)
    dst = edge_index[1].astype(jnp.int32)
    et = edge_type.astype(jnp.int32)
    pad = E_PAD - N_EDGES
    src = jnp.concatenate([src, jnp.zeros((pad,), jnp.int32)])
    dst = jnp.concatenate([dst, jnp.full((pad,), N_ENT, jnp.int32)])
    et = jnp.concatenate([et, jnp.zeros((pad,), jnp.int32)])
    out = _sc_aggregate(src, dst, et, initial_features, relation_embeddings)
    partial = out[:, :AGG_ROWS, :]
    degs_t = out[:, AGG_ROWS:, :].reshape(NW, DEG_ROWS * FEAT).T
    return _tc_dense(partial, degs_t, W_gcn, b_gcn.reshape(1, FEAT), Wp1,
                     bp1.reshape(1, FEAT), gamma.reshape(1, FEAT),
                     beta.reshape(1, FEAT), Wp2, bp2.reshape(1, FEAT))


# speedup vs baseline: 1.3548x; 1.3548x over previous
import functools

import jax
import jax.numpy as jnp
from jax import lax
from jax.experimental import pallas as pl
from jax.experimental.pallas import tpu as pltpu
from jax.experimental.pallas import tpu_sc as plsc

N_ENT = 10000
N_EDGES = 320000
FEAT = 128

NC = 2
NS = 16
NW = NC * NS
CHUNK = 64
N_CHUNKS = 158
EPW = CHUNK * N_CHUNKS
E_PAD = EPW * NW
AGG_ROWS = 10112
ROWS_PER_TILE = AGG_ROWS // NS
DEG_ROWS = 80
OUT_ROWS = AGG_ROWS + NS * DEG_ROWS


def _sc_body(idx_hbm, feat_hbm, rel_hbm, out_hbm,
             idx_v0, idx_v1,
             feat_v0, feat_v1, rel_v0, rel_v1, deg_v, agg_sh,
             sem_f0, sem_f1, sem_r0, sem_r1, sem_s0, sem_s1):
    cid = lax.axis_index("c")
    sid = lax.axis_index("s")
    w = cid * NS + sid

    idx_v = (idx_v0, idx_v1)
    feat_v = (feat_v0, feat_v1)
    rel_v = (rel_v0, rel_v1)
    sem_f = (sem_f0, sem_f1)
    sem_r = (sem_r0, sem_r1)
    sem_s = (sem_s0, sem_s1)

    zero16f = jnp.zeros((16,), jnp.float32)

    def zrow(r, _):
        for c in range(FEAT // 16):
            feat_v0[r, pl.ds(c * 16, 16)] = zero16f
        return 0
    lax.fori_loop(0, CHUNK, zrow, 0)
    base_row = sid * ROWS_PER_TILE
    for k in range(9):
        pltpu.sync_copy(feat_v0, agg_sh.at[pl.ds(base_row + k * CHUNK, CHUNK), :])
    pltpu.sync_copy(feat_v0.at[pl.ds(0, ROWS_PER_TILE - 9 * CHUNK), :],
                    agg_sh.at[pl.ds(base_row + 9 * CHUNK,
                                    ROWS_PER_TILE - 9 * CHUNK), :])

    def zdeg(r, _):
        for c in range(FEAT // 16):
            deg_v[r, pl.ds(c * 16, 16)] = zero16f
        return 0
    lax.fori_loop(0, DEG_ROWS, zdeg, 0)

    plsc.subcore_barrier()

    cbase = w * N_CHUNKS

    def stage_in(i, p):
        pltpu.sync_copy(idx_hbm.at[cbase + i], idx_v[p])
        pltpu.async_copy(feat_hbm.at[idx_v[p].at[0]], feat_v[p], sem_f[p])
        pltpu.async_copy(rel_hbm.at[idx_v[p].at[1]], rel_v[p], sem_r[p])

    def wait_gathers(p):
        pltpu.make_async_copy(feat_hbm.at[idx_v[p].at[0]], feat_v[p],
                              sem_f[p]).wait()
        pltpu.make_async_copy(rel_hbm.at[idx_v[p].at[1]], rel_v[p],
                              sem_r[p]).wait()

    def wait_scatter(p):
        pltpu.make_async_copy(feat_v[p], agg_sh.at[idx_v[p].at[2]],
                              sem_s[p]).wait()

    def work(i, p):
        wait_gathers(p)

        def mrow(j, _):
            r = j * 4
            for dr in range(4):
                for c in range(FEAT // 16):
                    feat_v[p][r + dr, pl.ds(c * 16, 16)] = (
                        feat_v[p][r + dr, pl.ds(c * 16, 16)]
                        * rel_v[p][r + dr, pl.ds(c * 16, 16)])
            return 0
        lax.fori_loop(0, CHUNK // 4, mrow, 0)

        pltpu.async_copy(feat_v[p], agg_sh.at[idx_v[p].at[2]], sem_s[p], add=True)

        def drow(j, _):
            d16 = idx_v[p][2, pl.ds(j * 16, 16)]
            cnt, last = plsc.scan_count(d16)
            plsc.addupdate_scatter(
                deg_v,
                [lax.shift_right_logical(d16, 7),
                 lax.bitwise_and(d16, 127)],
                cnt.astype(jnp.float32), mask=last)
            return 0
        lax.fori_loop(0, CHUNK // 16, drow, 0)

    stage_in(0, 0)

    def outer(j, _):
        i0 = j * 2
        for p in (0, 1):
            i = i0 + p
            q = 1 - p

            @pl.when(i > 0)
            def _():
                wait_scatter(q)

            @pl.when(i < N_CHUNKS - 1)
            def _():
                stage_in(i + 1, q)

            work(i, p)
        return 0

    lax.fori_loop(0, N_CHUNKS // 2, outer, 0)
    wait_scatter((N_CHUNKS - 1) % 2)

    plsc.subcore_barrier()

    pltpu.sync_copy(agg_sh.at[pl.ds(base_row, ROWS_PER_TILE), :],
                    out_hbm.at[cid, pl.ds(base_row, ROWS_PER_TILE), :])

    plsc.subcore_barrier()

    stage = sid * DEG_ROWS
    pltpu.sync_copy(deg_v, agg_sh.at[pl.ds(stage, DEG_ROWS), :])
    pltpu.sync_copy(agg_sh.at[pl.ds(stage, DEG_ROWS), :],
                    out_hbm.at[cid, pl.ds(AGG_ROWS + stage, DEG_ROWS), :])


_sc_aggregate = functools.partial(
    pl.kernel,
    out_type=jax.ShapeDtypeStruct((NC, OUT_ROWS, FEAT), jnp.float32),
    mesh=plsc.VectorSubcoreMesh(core_axis_name="c", subcore_axis_name="s"),
    compiler_params=pltpu.CompilerParams(needs_layout_passes=False),
    scratch_types=(
        [pltpu.VMEM((3, CHUNK), jnp.int32)] * 2
        + [pltpu.VMEM((CHUNK, FEAT), jnp.float32)] * 4
        + [pltpu.VMEM((DEG_ROWS, FEAT), jnp.float32),
           pltpu.VMEM_SHARED((AGG_ROWS, FEAT), jnp.float32)]
        + [pltpu.SemaphoreType.DMA] * 6
    ),
)(_sc_body)


def _tc_body(part_ref, degs_ref, Wg_ref, bg_ref, W1_ref, b1_ref, g_ref,
             be_ref, W2_ref, b2_ref, out_ref):
    agg = part_ref[0, :N_ENT, :] + part_ref[1, :N_ENT, :]
    deg = jnp.sum(degs_ref[...], axis=1, keepdims=True)[:N_ENT]
    h = agg / jnp.maximum(deg, 1.0)
    h = jnp.dot(h, Wg_ref[...], preferred_element_type=jnp.float32,
                precision=lax.Precision.HIGHEST) + bg_ref[...]
    z = jnp.dot(h, W1_ref[...], preferred_element_type=jnp.float32,
                precision=lax.Precision.HIGHEST) + b1_ref[...]
    mean = jnp.mean(z, axis=0, keepdims=True)
    var = jnp.mean(jnp.square(z - mean), axis=0, keepdims=True)
    zn = g_ref[...] * (z - mean) / jnp.sqrt(var + 1e-5) + be_ref[...]
    zr = jnp.maximum(zn, 0.0)
    out_ref[...] = jnp.dot(zr, W2_ref[...], preferred_element_type=jnp.float32,
                           precision=lax.Precision.HIGHEST) + b2_ref[...]


def _tc_dense(partial, degs_t, W_gcn, b_gcn, Wp1, bp1, gamma, beta, Wp2, bp2):
    return pl.pallas_call(
        _tc_body,
        out_shape=jax.ShapeDtypeStruct((N_ENT, FEAT), jnp.float32),
    )(partial, degs_t, W_gcn, b_gcn, Wp1, bp1, gamma, beta, Wp2, bp2)


def kernel(initial_features, relation_embeddings, W_gcn, b_gcn, Wp1, bp1,
           gamma, beta, Wp2, bp2, edge_index, edge_type):
    src = edge_index[0].astype(jnp.int32)
    dst = edge_index[1].astype(jnp.int32)
    et = edge_type.astype(jnp.int32)
    pad = E_PAD - N_EDGES
    src = jnp.concatenate([src, jnp.zeros((pad,), jnp.int32)])
    dst = jnp.concatenate([dst, jnp.full((pad,), N_ENT, jnp.int32)])
    et = jnp.concatenate([et, jnp.zeros((pad,), jnp.int32)])
    idx = jnp.stack([src.reshape(-1, CHUNK), et.reshape(-1, CHUNK),
                     dst.reshape(-1, CHUNK)], axis=1)
    out = _sc_aggregate(idx, initial_features, relation_embeddings)
    partial = out[:, :AGG_ROWS, :]
    degs_t = out[:, AGG_ROWS:, :].reshape(NW, DEG_ROWS * FEAT).T
    return _tc_dense(partial, degs_t, W_gcn, b_gcn.reshape(1, FEAT), Wp1,
                     bp1.reshape(1, FEAT), gamma.reshape(1, FEAT),
                     beta.reshape(1, FEAT), Wp2, bp2.reshape(1, FEAT))


# speedup vs baseline: 1.4879x; 1.0983x over previous
import functools

import jax
import jax.numpy as jnp
from jax import lax
from jax.experimental import pallas as pl
from jax.experimental.pallas import tpu as pltpu
from jax.experimental.pallas import tpu_sc as plsc

N_ENT = 10000
N_EDGES = 320000
FEAT = 128

NC = 2
NS = 16
NW = NC * NS
CHUNK = 64
N_CHUNKS = 316
CH_A = 188
CH_B = N_CHUNKS - CH_A
E_PAD = CHUNK * N_CHUNKS * NS
AGG_ROWS = 10112
ROWS_PER_TILE = AGG_ROWS // NS
DEG_ROWS = 80
OUT_ROWS = AGG_ROWS + NS * DEG_ROWS


def _sc_body(idx_hbm, feat_hbm, rel_hbm, out_hbm,
             idx_v0, idx_v1,
             feat_v0, feat_v1, rel_v0, rel_v1, deg_v, agg_sh,
             sem_f0, sem_f1, sem_r0, sem_r1, sem_s0, sem_s1):
    cid = lax.axis_index("c")
    sid = lax.axis_index("s")
    w = cid * NS + sid

    idx_v = (idx_v0, idx_v1)
    feat_v = (feat_v0, feat_v1)
    rel_v = (rel_v0, rel_v1)
    sem_f = (sem_f0, sem_f1)
    sem_r = (sem_r0, sem_r1)
    sem_s = (sem_s0, sem_s1)

    zero16f = jnp.zeros((16,), jnp.float32)

    def zrow(r, _):
        for c in range(FEAT // 16):
            feat_v0[r, pl.ds(c * 16, 16)] = zero16f
        return 0
    lax.fori_loop(0, CHUNK, zrow, 0)
    base_row = sid * ROWS_PER_TILE
    for k in range(9):
        pltpu.sync_copy(feat_v0, agg_sh.at[pl.ds(base_row + k * CHUNK, CHUNK), :])
    pltpu.sync_copy(feat_v0.at[pl.ds(0, ROWS_PER_TILE - 9 * CHUNK), :],
                    agg_sh.at[pl.ds(base_row + 9 * CHUNK,
                                    ROWS_PER_TILE - 9 * CHUNK), :])

    def zdeg(r, _):
        for c in range(FEAT // 16):
            deg_v[r, pl.ds(c * 16, 16)] = zero16f
        return 0
    lax.fori_loop(0, DEG_ROWS, zdeg, 0)

    plsc.subcore_barrier()

    nch = lax.select(cid == 0, jnp.int32(CH_A), jnp.int32(CH_B))
    cbase = lax.select(cid == 0, sid * CH_A, NS * CH_A + sid * CH_B)

    def stage_in(i, p):
        pltpu.sync_copy(idx_hbm.at[cbase + i], idx_v[p])
        pltpu.async_copy(feat_hbm.at[idx_v[p].at[0]], feat_v[p], sem_f[p])
        pltpu.async_copy(rel_hbm.at[idx_v[p].at[1]], rel_v[p], sem_r[p])

    def wait_gathers(p):
        pltpu.make_async_copy(feat_hbm.at[idx_v[p].at[0]], feat_v[p],
                              sem_f[p]).wait()
        pltpu.make_async_copy(rel_hbm.at[idx_v[p].at[1]], rel_v[p],
                              sem_r[p]).wait()

    def wait_scatter(p):
        pltpu.make_async_copy(feat_v[p], agg_sh.at[idx_v[p].at[2]],
                              sem_s[p]).wait()

    def work(i, p):
        wait_gathers(p)

        def mrow(j, _):
            r = j * 4
            for dr in range(4):
                for c in range(FEAT // 16):
                    feat_v[p][r + dr, pl.ds(c * 16, 16)] = (
                        feat_v[p][r + dr, pl.ds(c * 16, 16)]
                        * rel_v[p][r + dr, pl.ds(c * 16, 16)])
            return 0
        lax.fori_loop(0, CHUNK // 4, mrow, 0)

        pltpu.async_copy(feat_v[p], agg_sh.at[idx_v[p].at[2]], sem_s[p], add=True)

        def drow(j, _):
            d16 = idx_v[p][2, pl.ds(j * 16, 16)]
            cnt, last = plsc.scan_count(d16)
            plsc.addupdate_scatter(
                deg_v,
                [lax.shift_right_logical(d16, 7),
                 lax.bitwise_and(d16, 127)],
                cnt.astype(jnp.float32), mask=last)
            return 0
        lax.fori_loop(0, CHUNK // 16, drow, 0)

    stage_in(0, 0)

    def outer(j, _):
        i0 = j * 2
        for p in (0, 1):
            i = i0 + p
            q = 1 - p

            @pl.when(i > 0)
            def _():
                wait_scatter(q)

            @pl.when(i < nch - 1)
            def _():
                stage_in(i + 1, q)

            work(i, p)
        return 0

    lax.fori_loop(0, lax.div(nch, jnp.int32(2)), outer, 0)
    wait_scatter(1)

    plsc.subcore_barrier()

    pltpu.sync_copy(agg_sh.at[pl.ds(base_row, ROWS_PER_TILE), :],
                    out_hbm.at[cid, pl.ds(base_row, ROWS_PER_TILE), :])

    plsc.subcore_barrier()

    stage = sid * DEG_ROWS
    pltpu.sync_copy(deg_v, agg_sh.at[pl.ds(stage, DEG_ROWS), :])
    pltpu.sync_copy(agg_sh.at[pl.ds(stage, DEG_ROWS), :],
                    out_hbm.at[cid, pl.ds(AGG_ROWS + stage, DEG_ROWS), :])


_sc_aggregate = functools.partial(
    pl.kernel,
    out_type=jax.ShapeDtypeStruct((NC, OUT_ROWS, FEAT), jnp.float32),
    mesh=plsc.VectorSubcoreMesh(core_axis_name="c", subcore_axis_name="s"),
    compiler_params=pltpu.CompilerParams(needs_layout_passes=False),
    scratch_types=(
        [pltpu.VMEM((3, CHUNK), jnp.int32)] * 2
        + [pltpu.VMEM((CHUNK, FEAT), jnp.float32)] * 4
        + [pltpu.VMEM((DEG_ROWS, FEAT), jnp.float32),
           pltpu.VMEM_SHARED((AGG_ROWS, FEAT), jnp.float32)]
        + [pltpu.SemaphoreType.DMA] * 6
    ),
)(_sc_body)


def _tc_body(part_ref, degs_ref, Wg_ref, bg_ref, W1_ref, b1_ref, g_ref,
             be_ref, W2_ref, b2_ref, out_ref):
    agg = part_ref[0, :N_ENT, :] + part_ref[1, :N_ENT, :]
    deg = jnp.sum(degs_ref[...], axis=1, keepdims=True)[:N_ENT]
    h = agg / jnp.maximum(deg, 1.0)
    h = jnp.dot(h, Wg_ref[...], preferred_element_type=jnp.float32,
                precision=lax.Precision.HIGHEST) + bg_ref[...]
    z = jnp.dot(h, W1_ref[...], preferred_element_type=jnp.float32,
                precision=lax.Precision.HIGHEST) + b1_ref[...]
    mean = jnp.mean(z, axis=0, keepdims=True)
    var = jnp.mean(jnp.square(z - mean), axis=0, keepdims=True)
    zn = g_ref[...] * (z - mean) / jnp.sqrt(var + 1e-5) + be_ref[...]
    zr = jnp.maximum(zn, 0.0)
    out_ref[...] = jnp.dot(zr, W2_ref[...], preferred_element_type=jnp.float32,
                           precision=lax.Precision.HIGHEST) + b2_ref[...]


def _tc_dense(partial, degs_t, W_gcn, b_gcn, Wp1, bp1, gamma, beta, Wp2, bp2):
    return pl.pallas_call(
        _tc_body,
        out_shape=jax.ShapeDtypeStruct((N_ENT, FEAT), jnp.float32),
    )(partial, degs_t, W_gcn, b_gcn, Wp1, bp1, gamma, beta, Wp2, bp2)


def kernel(initial_features, relation_embeddings, W_gcn, b_gcn, Wp1, bp1,
           gamma, beta, Wp2, bp2, edge_index, edge_type):
    src = edge_index[0].astype(jnp.int32)
    dst = edge_index[1].astype(jnp.int32)
    et = edge_type.astype(jnp.int32)
    pad = E_PAD - N_EDGES
    src = jnp.concatenate([src, jnp.zeros((pad,), jnp.int32)])
    dst = jnp.concatenate([dst, jnp.full((pad,), N_ENT, jnp.int32)])
    et = jnp.concatenate([et, jnp.zeros((pad,), jnp.int32)])
    idx = jnp.stack([src.reshape(-1, CHUNK), et.reshape(-1, CHUNK),
                     dst.reshape(-1, CHUNK)], axis=1)
    out = _sc_aggregate(idx, initial_features, relation_embeddings)
    partial = out[:, :AGG_ROWS, :]
    degs_t = out[:, AGG_ROWS:, :].reshape(NW, DEG_ROWS * FEAT).T
    return _tc_dense(partial, degs_t, W_gcn, b_gcn.reshape(1, FEAT), Wp1,
                     bp1.reshape(1, FEAT), gamma.reshape(1, FEAT),
                     beta.reshape(1, FEAT), Wp2, bp2.reshape(1, FEAT))


# speedup vs baseline: 1.5224x; 1.0232x over previous
import functools

import jax
import jax.numpy as jnp
from jax import lax
from jax.experimental import pallas as pl
from jax.experimental.pallas import tpu as pltpu
from jax.experimental.pallas import tpu_sc as plsc

N_ENT = 10000
N_EDGES = 320000
FEAT = 128

NC = 2
NS = 16
NW = NC * NS
CHUNK = 64
N_CHUNKS = 316
CH_A = 196
CH_B = N_CHUNKS - CH_A
E_PAD = CHUNK * N_CHUNKS * NS
AGG_ROWS = 10112
ROWS_PER_TILE = AGG_ROWS // NS
DEG_ROWS = 80
OUT_ROWS = AGG_ROWS + NS * DEG_ROWS


def _sc_body(idx_hbm, feat_hbm, rel_hbm, out_hbm,
             idx_v0, idx_v1,
             feat_v0, feat_v1, rel_v0, rel_v1, deg_v, agg_sh,
             sem_f0, sem_f1, sem_r0, sem_r1, sem_s0, sem_s1):
    cid = lax.axis_index("c")
    sid = lax.axis_index("s")
    w = cid * NS + sid

    idx_v = (idx_v0, idx_v1)
    feat_v = (feat_v0, feat_v1)
    rel_v = (rel_v0, rel_v1)
    sem_f = (sem_f0, sem_f1)
    sem_r = (sem_r0, sem_r1)
    sem_s = (sem_s0, sem_s1)

    zero16f = jnp.zeros((16,), jnp.float32)

    def zrow(r, _):
        for c in range(FEAT // 16):
            feat_v0[r, pl.ds(c * 16, 16)] = zero16f
        return 0
    lax.fori_loop(0, CHUNK, zrow, 0)
    base_row = sid * ROWS_PER_TILE
    for k in range(9):
        pltpu.sync_copy(feat_v0, agg_sh.at[pl.ds(base_row + k * CHUNK, CHUNK), :])
    pltpu.sync_copy(feat_v0.at[pl.ds(0, ROWS_PER_TILE - 9 * CHUNK), :],
                    agg_sh.at[pl.ds(base_row + 9 * CHUNK,
                                    ROWS_PER_TILE - 9 * CHUNK), :])

    def zdeg(r, _):
        for c in range(FEAT // 16):
            deg_v[r, pl.ds(c * 16, 16)] = zero16f
        return 0
    lax.fori_loop(0, DEG_ROWS, zdeg, 0)

    plsc.subcore_barrier()

    nch = lax.select(cid == 0, jnp.int32(CH_A), jnp.int32(CH_B))
    cbase = lax.select(cid == 0, sid * CH_A, NS * CH_A + sid * CH_B)

    def stage_in(i, p):
        pltpu.sync_copy(idx_hbm.at[cbase + i], idx_v[p])
        pltpu.async_copy(feat_hbm.at[idx_v[p].at[0]], feat_v[p], sem_f[p])
        pltpu.async_copy(rel_hbm.at[idx_v[p].at[1]], rel_v[p], sem_r[p])

    def wait_gathers(p):
        pltpu.make_async_copy(feat_hbm.at[idx_v[p].at[0]], feat_v[p],
                              sem_f[p]).wait()
        pltpu.make_async_copy(rel_hbm.at[idx_v[p].at[1]], rel_v[p],
                              sem_r[p]).wait()

    def wait_scatter(p):
        pltpu.make_async_copy(feat_v[p], agg_sh.at[idx_v[p].at[2]],
                              sem_s[p]).wait()

    def work(i, p):
        wait_gathers(p)

        def mrow(j, _):
            r = j * 8
            for dr in range(8):
                for c in range(FEAT // 16):
                    feat_v[p][r + dr, pl.ds(c * 16, 16)] = (
                        feat_v[p][r + dr, pl.ds(c * 16, 16)]
                        * rel_v[p][r + dr, pl.ds(c * 16, 16)])
            return 0
        lax.fori_loop(0, CHUNK // 8, mrow, 0)

        pltpu.async_copy(feat_v[p], agg_sh.at[idx_v[p].at[2]], sem_s[p], add=True)

        def drow(j, _):
            d16 = idx_v[p][2, pl.ds(j * 16, 16)]
            cnt, last = plsc.scan_count(d16)
            plsc.addupdate_scatter(
                deg_v,
                [lax.shift_right_logical(d16, 7),
                 lax.bitwise_and(d16, 127)],
                cnt.astype(jnp.float32), mask=last)
            return 0
        lax.fori_loop(0, CHUNK // 16, drow, 0)

    stage_in(0, 0)

    def outer(j, _):
        i0 = j * 2
        for p in (0, 1):
            i = i0 + p
            q = 1 - p

            @pl.when(i > 0)
            def _():
                wait_scatter(q)

            @pl.when(i < nch - 1)
            def _():
                stage_in(i + 1, q)

            work(i, p)
        return 0

    lax.fori_loop(0, lax.div(nch, jnp.int32(2)), outer, 0)
    wait_scatter(1)

    plsc.subcore_barrier()

    pltpu.sync_copy(agg_sh.at[pl.ds(base_row, ROWS_PER_TILE), :],
                    out_hbm.at[cid, pl.ds(base_row, ROWS_PER_TILE), :])

    plsc.subcore_barrier()

    stage = sid * DEG_ROWS
    pltpu.sync_copy(deg_v, agg_sh.at[pl.ds(stage, DEG_ROWS), :])
    pltpu.sync_copy(agg_sh.at[pl.ds(stage, DEG_ROWS), :],
                    out_hbm.at[cid, pl.ds(AGG_ROWS + stage, DEG_ROWS), :])


_sc_aggregate = functools.partial(
    pl.kernel,
    out_type=jax.ShapeDtypeStruct((NC, OUT_ROWS, FEAT), jnp.float32),
    mesh=plsc.VectorSubcoreMesh(core_axis_name="c", subcore_axis_name="s"),
    compiler_params=pltpu.CompilerParams(needs_layout_passes=False),
    scratch_types=(
        [pltpu.VMEM((3, CHUNK), jnp.int32)] * 2
        + [pltpu.VMEM((CHUNK, FEAT), jnp.float32)] * 4
        + [pltpu.VMEM((DEG_ROWS, FEAT), jnp.float32),
           pltpu.VMEM_SHARED((AGG_ROWS, FEAT), jnp.float32)]
        + [pltpu.SemaphoreType.DMA] * 6
    ),
)(_sc_body)


def _tc_body(part_ref, degs_ref, Wg_ref, bg_ref, W1_ref, b1_ref, g_ref,
             be_ref, W2_ref, b2_ref, out_ref):
    agg = part_ref[0, :N_ENT, :] + part_ref[1, :N_ENT, :]
    deg = jnp.sum(degs_ref[...], axis=1, keepdims=True)[:N_ENT]
    h = agg / jnp.maximum(deg, 1.0)
    h = jnp.dot(h, Wg_ref[...], preferred_element_type=jnp.float32,
                precision=lax.Precision.HIGHEST) + bg_ref[...]
    z = jnp.dot(h, W1_ref[...], preferred_element_type=jnp.float32,
                precision=lax.Precision.HIGHEST) + b1_ref[...]
    mean = jnp.mean(z, axis=0, keepdims=True)
    var = jnp.mean(jnp.square(z - mean), axis=0, keepdims=True)
    zn = g_ref[...] * (z - mean) / jnp.sqrt(var + 1e-5) + be_ref[...]
    zr = jnp.maximum(zn, 0.0)
    out_ref[...] = jnp.dot(zr, W2_ref[...], preferred_element_type=jnp.float32,
                           precision=lax.Precision.HIGHEST) + b2_ref[...]


def _tc_dense(partial, degs_t, W_gcn, b_gcn, Wp1, bp1, gamma, beta, Wp2, bp2):
    return pl.pallas_call(
        _tc_body,
        out_shape=jax.ShapeDtypeStruct((N_ENT, FEAT), jnp.float32),
    )(partial, degs_t, W_gcn, b_gcn, Wp1, bp1, gamma, beta, Wp2, bp2)


def kernel(initial_features, relation_embeddings, W_gcn, b_gcn, Wp1, bp1,
           gamma, beta, Wp2, bp2, edge_index, edge_type):
    src = edge_index[0].astype(jnp.int32)
    dst = edge_index[1].astype(jnp.int32)
    et = edge_type.astype(jnp.int32)
    pad = E_PAD - N_EDGES
    src = jnp.concatenate([src, jnp.zeros((pad,), jnp.int32)])
    dst = jnp.concatenate([dst, jnp.full((pad,), N_ENT, jnp.int32)])
    et = jnp.concatenate([et, jnp.zeros((pad,), jnp.int32)])
    idx = jnp.stack([src.reshape(-1, CHUNK), et.reshape(-1, CHUNK),
                     dst.reshape(-1, CHUNK)], axis=1)
    out = _sc_aggregate(idx, initial_features, relation_embeddings)
    partial = out[:, :AGG_ROWS, :]
    degs_t = out[:, AGG_ROWS:, :].reshape(NW, DEG_ROWS * FEAT).T
    return _tc_dense(partial, degs_t, W_gcn, b_gcn.reshape(1, FEAT), Wp1,
                     bp1.reshape(1, FEAT), gamma.reshape(1, FEAT),
                     beta.reshape(1, FEAT), Wp2, bp2.reshape(1, FEAT))
